# baseline (speedup 1.0000x reference)
import dataclasses
import functools

import jax
import jax.numpy as jnp
from jax import lax
from jax.experimental import pallas as pl
from jax.experimental.pallas import tpu as pltpu
from jax.experimental.pallas import tpu_sc as plsc

_cp = pltpu.CompilerParams()
if "needs_layout_passes" in pltpu.CompilerParams.__dataclass_fields__:
    _cp = dataclasses.replace(_cp, needs_layout_passes=False)

N = 10000
E = 320000
D = 128

NC = 2
NS = 16
NW = NC * NS
L = 16

NP = 10240
EPW = E // NW
C = 125
NCHUNK = EPW // C
RPS = NP // NS
ZR = 128
KF = 8

_mesh = plsc.VectorSubcoreMesh(core_axis_name="c", subcore_axis_name="s")


def _zero_fill(buf, nrows, ncols):
    z16 = jnp.zeros((16,), jnp.float32)

    @pl.loop(0, nrows)
    def _(r):
        @pl.loop(0, ncols, step=16)
        def _(c0):
            buf[r, pl.ds(c0, 16)] = z16


def _sc_degrees(src, dst):
    @functools.partial(
        pl.kernel,
        out_type=(
            jax.ShapeDtypeStruct((NC, NP), jnp.float32),
            jax.ShapeDtypeStruct((NC, NP), jnp.float32),
        ),
        mesh=_mesh,
        scratch_types=[
            pltpu.VMEM_SHARED((NS, NP), jnp.float32),
            pltpu.VMEM((NP,), jnp.float32),
            pltpu.VMEM((NP,), jnp.float32),
            pltpu.VMEM((EPW,), jnp.int32),
            pltpu.VMEM((EPW,), jnp.int32),
            pltpu.VMEM((NS, RPS), jnp.float32),
            pltpu.VMEM((RPS,), jnp.float32),
        ],
        compiler_params=_cp,
    )
    def deg_kernel(src_hbm, dst_hbm, do_hbm, di_hbm, sh2, ho, hi, sidx,
                   didx, red, outv):
        cid = lax.axis_index("c")
        sid = lax.axis_index("s")
        wid = sid * NC + cid
        z16 = jnp.zeros((16,), jnp.float32)
        o16 = jnp.ones((16,), jnp.float32)

        @pl.loop(0, NP, step=16)
        def _(i):
            ho[pl.ds(i, 16)] = z16
            hi[pl.ds(i, 16)] = z16

        pltpu.sync_copy(src_hbm.at[pl.ds(wid * EPW, EPW)], sidx)
        pltpu.sync_copy(dst_hbm.at[pl.ds(wid * EPW, EPW)], didx)

        @pl.loop(0, EPW, step=16)
        def _(j):
            plsc.addupdate_scatter(ho, [sidx[pl.ds(j, 16)]], o16)
            plsc.addupdate_scatter(hi, [didx[pl.ds(j, 16)]], o16)

        for h, out_hbm in ((ho, do_hbm), (hi, di_hbm)):
            pltpu.sync_copy(h, sh2.at[sid])
            plsc.subcore_barrier()

            @pl.loop(0, NS)
            def _(t):
                pltpu.sync_copy(sh2.at[t, pl.ds(sid * RPS, RPS)], red.at[t])

            @pl.loop(0, RPS, step=16)
            def _(c0):
                acc = red[0, pl.ds(c0, 16)]
                for t in range(1, NS):
                    acc = acc + red[t, pl.ds(c0, 16)]
                outv[pl.ds(c0, 16)] = acc

            pltpu.sync_copy(outv, out_hbm.at[cid, pl.ds(sid * RPS, RPS)])
            plsc.subcore_barrier()

    return deg_kernel(src, dst)


def _sc_aggregate(h, src, dst):
    @functools.partial(
        pl.kernel,
        out_type=jax.ShapeDtypeStruct((NC, NP, D), jnp.float32),
        mesh=_mesh,
        scratch_types=[
            pltpu.VMEM_SHARED((NP, D), jnp.float32),
            pltpu.VMEM((C, D), jnp.float32),
            pltpu.VMEM((C, D), jnp.float32),
            pltpu.VMEM((NCHUNK, C), jnp.int32),
            pltpu.VMEM((2, C), jnp.int32),
            pltpu.SemaphoreType.DMA,
            pltpu.SemaphoreType.DMA,
            pltpu.SemaphoreType.DMA,
            pltpu.SemaphoreType.DMA,
        ],
    )
    def agg_kernel(h_hbm, src_hbm, dst_hbm, out_hbm, sh_agg, rows0, rows1,
                   sidx2, didx, gsem0, gsem1, dsem0, dsem1):
        cid = lax.axis_index("c")
        sid = lax.axis_index("s")
        wid = sid * NC + cid

        _zero_fill(rows0, C, D)
        pltpu.sync_copy(src_hbm.at[wid], sidx2)

        @pl.loop(0, 600, step=120)
        def _(j):
            pltpu.sync_copy(rows0.at[pl.ds(0, 120)],
                            sh_agg.at[pl.ds(sid * RPS + j, 120)])

        pltpu.sync_copy(rows0.at[pl.ds(0, 40)],
                        sh_agg.at[pl.ds(sid * RPS + 600, 40)])

        plsc.subcore_barrier()

        CH = 64

        def _gather(k, rows, sem):
            pltpu.async_copy(h_hbm.at[sidx2.at[k, pl.ds(0, CH)]],
                             rows.at[pl.ds(0, CH)], sem)
            pltpu.async_copy(h_hbm.at[sidx2.at[k, pl.ds(CH, C - CH)]],
                             rows.at[pl.ds(CH, C - CH)], sem)

        def _gather_wait(k, rows, sem):
            pltpu.make_async_copy(h_hbm.at[sidx2.at[k, pl.ds(0, CH)]],
                                  rows.at[pl.ds(0, CH)], sem).wait()
            pltpu.make_async_copy(h_hbm.at[sidx2.at[k, pl.ds(CH, C - CH)]],
                                  rows.at[pl.ds(CH, C - CH)], sem).wait()

        _gather(0, rows0, gsem0)
        _gather(1, rows1, gsem1)
        pltpu.async_copy(dst_hbm.at[wid, 0], didx.at[0], dsem0)
        pltpu.async_copy(dst_hbm.at[wid, 1], didx.at[1], dsem1)

        @pl.loop(0, NCHUNK, step=2)
        def _(k):
            _gather_wait(k, rows0, gsem0)
            pltpu.make_async_copy(dst_hbm.at[wid, k], didx.at[0],
                                  dsem0).wait()
            pltpu.sync_copy(rows0, sh_agg.at[didx.at[0]], add=True)

            @pl.when(k + 2 < NCHUNK)
            def _():
                _gather(k + 2, rows0, gsem0)
                pltpu.async_copy(dst_hbm.at[wid, k + 2], didx.at[0], dsem0)

            _gather_wait(k + 1, rows1, gsem1)
            pltpu.make_async_copy(dst_hbm.at[wid, k + 1], didx.at[1],
                                  dsem1).wait()
            pltpu.sync_copy(rows1, sh_agg.at[didx.at[1]], add=True)

            @pl.when(k + 3 < NCHUNK)
            def _():
                _gather(k + 3, rows1, gsem1)
                pltpu.async_copy(dst_hbm.at[wid, k + 3], didx.at[1], dsem1)

        plsc.subcore_barrier()

        sl = pl.ds(sid * RPS, RPS)
        pltpu.sync_copy(sh_agg.at[sl], out_hbm.at[cid, sl])

    return agg_kernel(h, src.reshape(NW, NCHUNK, C),
                      dst.reshape(NW, NCHUNK, C))


_R = 1000


def _tc_norms(deg_out, deg_in, x):
    def body(co_ref, ci_ref, x_ref, xs_ref, ns_ref, nd_ref):
        deg_o = co_ref[0] + co_ref[1]
        deg_i = ci_ref[0] + ci_ref[1]
        ns = jnp.where(deg_o > 0, lax.rsqrt(jnp.maximum(deg_o, 1e-12)), 0.0)
        nd = jnp.where(deg_i > 0, lax.rsqrt(jnp.maximum(deg_i, 1e-12)), 0.0)
        ns_ref[...] = ns
        nd_ref[...] = nd
        xs_ref[...] = x_ref[...] * ns

    return pl.pallas_call(
        body,
        grid=(N // _R,),
        in_specs=[
            pl.BlockSpec((NC, _R, 1), lambda i: (0, i, 0)),
            pl.BlockSpec((NC, _R, 1), lambda i: (0, i, 0)),
            pl.BlockSpec((_R, D), lambda i: (i, 0)),
        ],
        out_specs=[
            pl.BlockSpec((_R, D), lambda i: (i, 0)),
            pl.BlockSpec((_R, 1), lambda i: (i, 0)),
            pl.BlockSpec((_R, 1), lambda i: (i, 0)),
        ],
        out_shape=[
            jax.ShapeDtypeStruct((N, D), jnp.float32),
            jax.ShapeDtypeStruct((N, 1), jnp.float32),
            jax.ShapeDtypeStruct((N, 1), jnp.float32),
        ],
    )(deg_out.reshape(NC, NP, 1), deg_in.reshape(NC, NP, 1), x)


def _tc_layer(agg, nd, ns, w, b, scale_out):
    def body(a_ref, nd_ref, ns_ref, w_ref, b_ref, o_ref):
        a = (a_ref[0] + a_ref[1]) * nd_ref[...]
        h = jnp.dot(a, w_ref[...], preferred_element_type=jnp.float32)
        h = jnp.maximum(h + b_ref[...], 0.0)
        if scale_out:
            h = h * ns_ref[...]
        o_ref[...] = h

    return pl.pallas_call(
        body,
        grid=(N // _R,),
        in_specs=[
            pl.BlockSpec((NC, _R, D), lambda i: (0, i, 0)),
            pl.BlockSpec((_R, 1), lambda i: (i, 0)),
            pl.BlockSpec((_R, 1), lambda i: (i, 0)),
            pl.BlockSpec((D, D), lambda i: (0, 0)),
            pl.BlockSpec((1, D), lambda i: (0, 0)),
        ],
        out_specs=pl.BlockSpec((_R, D), lambda i: (i, 0)),
        out_shape=jax.ShapeDtypeStruct((N, D), jnp.float32),
    )(agg, nd, ns, w, b)


def kernel(x, edge_index, W1, b1, W2, b2):
    src = edge_index[0].astype(jnp.int32)
    dst = edge_index[1].astype(jnp.int32)

    deg_out, deg_in = _sc_degrees(src, dst)
    xs, nsrc, ndst = _tc_norms(deg_out, deg_in, x)

    agg1 = _sc_aggregate(xs, src, dst)
    h1 = _tc_layer(agg1, ndst, nsrc, W1, b1.reshape(1, D), scale_out=True)

    agg2 = _sc_aggregate(h1, src, dst)
    out = _tc_layer(agg2, ndst, nsrc, W2, b2.reshape(1, D), scale_out=False)
    return out


# speedup vs baseline: 1.0167x; 1.0167x over previous
import dataclasses
import functools

import jax
import jax.numpy as jnp
from jax import lax
from jax.experimental import pallas as pl
from jax.experimental.pallas import tpu as pltpu
from jax.experimental.pallas import tpu_sc as plsc

_cp = pltpu.CompilerParams()
if "needs_layout_passes" in pltpu.CompilerParams.__dataclass_fields__:
    _cp = dataclasses.replace(_cp, needs_layout_passes=False)

N = 10000
E = 320000
D = 128

NC = 2
NS = 16
NW = NC * NS
L = 16

NP = 10240
EPW = E // NW
C = 125
NCHUNK = EPW // C
RPS = NP // NS
ZR = 128
KF = 8

_mesh = plsc.VectorSubcoreMesh(core_axis_name="c", subcore_axis_name="s")


def _zero_fill(buf, nrows, ncols):
    z16 = jnp.zeros((16,), jnp.float32)

    @pl.loop(0, nrows)
    def _(r):
        @pl.loop(0, ncols, step=16)
        def _(c0):
            buf[r, pl.ds(c0, 16)] = z16


def _sc_degrees(src, dst):
    @functools.partial(
        pl.kernel,
        out_type=(
            jax.ShapeDtypeStruct((NC, NP), jnp.float32),
            jax.ShapeDtypeStruct((NC, NP), jnp.float32),
        ),
        mesh=_mesh,
        scratch_types=[
            pltpu.VMEM_SHARED((NS, NP), jnp.float32),
            pltpu.VMEM_SHARED((NS, NP), jnp.float32),
            pltpu.VMEM((NP,), jnp.float32),
            pltpu.VMEM((NP,), jnp.float32),
            pltpu.VMEM((EPW,), jnp.int32),
            pltpu.VMEM((EPW,), jnp.int32),
            pltpu.VMEM((NS, RPS), jnp.float32),
            pltpu.VMEM((RPS,), jnp.float32),
        ],
        compiler_params=_cp,
    )
    def deg_kernel(src_hbm, dst_hbm, do_hbm, di_hbm, sh2o, sh2i, ho, hi,
                   sidx, didx, red, outv):
        cid = lax.axis_index("c")
        sid = lax.axis_index("s")
        wid = sid * NC + cid
        z16 = jnp.zeros((16,), jnp.float32)
        o16 = jnp.ones((16,), jnp.float32)

        @pl.loop(0, NP, step=16)
        def _(i):
            ho[pl.ds(i, 16)] = z16
            hi[pl.ds(i, 16)] = z16

        pltpu.sync_copy(src_hbm.at[pl.ds(wid * EPW, EPW)], sidx)
        pltpu.sync_copy(dst_hbm.at[pl.ds(wid * EPW, EPW)], didx)

        @pl.loop(0, EPW, step=16)
        def _(j):
            plsc.addupdate_scatter(ho, [sidx[pl.ds(j, 16)]], o16)
            plsc.addupdate_scatter(hi, [didx[pl.ds(j, 16)]], o16)

        pltpu.sync_copy(ho, sh2o.at[sid])
        pltpu.sync_copy(hi, sh2i.at[sid])
        plsc.subcore_barrier()

        for sh2, out_hbm in ((sh2o, do_hbm), (sh2i, di_hbm)):
            @pl.loop(0, NS)
            def _(t):
                pltpu.sync_copy(sh2.at[t, pl.ds(sid * RPS, RPS)], red.at[t])

            @pl.loop(0, RPS, step=16)
            def _(c0):
                acc = red[0, pl.ds(c0, 16)]
                for t in range(1, NS):
                    acc = acc + red[t, pl.ds(c0, 16)]
                outv[pl.ds(c0, 16)] = acc

            pltpu.sync_copy(outv, out_hbm.at[cid, pl.ds(sid * RPS, RPS)])

    return deg_kernel(src, dst)


def _sc_aggregate(h, src, dst):
    @functools.partial(
        pl.kernel,
        out_type=jax.ShapeDtypeStruct((NC, NP, D), jnp.float32),
        mesh=_mesh,
        scratch_types=[
            pltpu.VMEM_SHARED((NP, D), jnp.float32),
            pltpu.VMEM((C, D), jnp.float32),
            pltpu.VMEM((C, D), jnp.float32),
            pltpu.VMEM((NCHUNK, C), jnp.int32),
            pltpu.VMEM((2, C), jnp.int32),
            pltpu.SemaphoreType.DMA,
            pltpu.SemaphoreType.DMA,
            pltpu.SemaphoreType.DMA,
            pltpu.SemaphoreType.DMA,
        ],
    )
    def agg_kernel(h_hbm, src_hbm, dst_hbm, out_hbm, sh_agg, rows0, rows1,
                   sidx2, didx, gsem0, gsem1, dsem0, dsem1):
        cid = lax.axis_index("c")
        sid = lax.axis_index("s")
        wid = sid * NC + cid

        pltpu.sync_copy(src_hbm.at[wid], sidx2)
        pltpu.async_copy(h_hbm.at[sidx2.at[0]], rows0, gsem0)
        pltpu.async_copy(dst_hbm.at[wid, 0], didx.at[0], dsem0)
        pltpu.async_copy(dst_hbm.at[wid, 1], didx.at[1], dsem1)

        _zero_fill(rows1, C, D)

        @pl.loop(0, 600, step=120)
        def _(j):
            pltpu.sync_copy(rows1.at[pl.ds(0, 120)],
                            sh_agg.at[pl.ds(sid * RPS + j, 120)])

        pltpu.sync_copy(rows1.at[pl.ds(0, 40)],
                        sh_agg.at[pl.ds(sid * RPS + 600, 40)])

        pltpu.async_copy(h_hbm.at[sidx2.at[1]], rows1, gsem1)
        plsc.subcore_barrier()

        @pl.loop(0, NCHUNK, step=2)
        def _(k):
            pltpu.make_async_copy(h_hbm.at[sidx2.at[k]], rows0, gsem0).wait()
            pltpu.make_async_copy(dst_hbm.at[wid, k], didx.at[0],
                                  dsem0).wait()
            pltpu.sync_copy(rows0, sh_agg.at[didx.at[0]], add=True)

            @pl.when(k + 2 < NCHUNK)
            def _():
                pltpu.async_copy(h_hbm.at[sidx2.at[k + 2]], rows0, gsem0)
                pltpu.async_copy(dst_hbm.at[wid, k + 2], didx.at[0], dsem0)

            pltpu.make_async_copy(h_hbm.at[sidx2.at[k + 1]], rows1,
                                  gsem1).wait()
            pltpu.make_async_copy(dst_hbm.at[wid, k + 1], didx.at[1],
                                  dsem1).wait()
            pltpu.sync_copy(rows1, sh_agg.at[didx.at[1]], add=True)

            @pl.when(k + 3 < NCHUNK)
            def _():
                pltpu.async_copy(h_hbm.at[sidx2.at[k + 3]], rows1, gsem1)
                pltpu.async_copy(dst_hbm.at[wid, k + 3], didx.at[1], dsem1)

        plsc.subcore_barrier()

        sl = pl.ds(sid * RPS, RPS)
        pltpu.sync_copy(sh_agg.at[sl], out_hbm.at[cid, sl])

    return agg_kernel(h, src.reshape(NW, NCHUNK, C),
                      dst.reshape(NW, NCHUNK, C))


_R = 1000


def _tc_norms(deg_out, deg_in, x):
    def body(co_ref, ci_ref, x_ref, xs_ref, ns_ref, nd_ref):
        deg_o = co_ref[0] + co_ref[1]
        deg_i = ci_ref[0] + ci_ref[1]
        ns = jnp.where(deg_o > 0, lax.rsqrt(jnp.maximum(deg_o, 1e-12)), 0.0)
        nd = jnp.where(deg_i > 0, lax.rsqrt(jnp.maximum(deg_i, 1e-12)), 0.0)
        ns_ref[...] = ns
        nd_ref[...] = nd
        xs_ref[...] = x_ref[...] * ns

    return pl.pallas_call(
        body,
        grid=(N // _R,),
        in_specs=[
            pl.BlockSpec((NC, _R, 1), lambda i: (0, i, 0)),
            pl.BlockSpec((NC, _R, 1), lambda i: (0, i, 0)),
            pl.BlockSpec((_R, D), lambda i: (i, 0)),
        ],
        out_specs=[
            pl.BlockSpec((_R, D), lambda i: (i, 0)),
            pl.BlockSpec((_R, 1), lambda i: (i, 0)),
            pl.BlockSpec((_R, 1), lambda i: (i, 0)),
        ],
        out_shape=[
            jax.ShapeDtypeStruct((N, D), jnp.float32),
            jax.ShapeDtypeStruct((N, 1), jnp.float32),
            jax.ShapeDtypeStruct((N, 1), jnp.float32),
        ],
    )(deg_out.reshape(NC, NP, 1), deg_in.reshape(NC, NP, 1), x)


def _tc_layer(agg, nd, ns, w, b, scale_out):
    def body(a_ref, nd_ref, ns_ref, w_ref, b_ref, o_ref):
        a = (a_ref[0] + a_ref[1]) * nd_ref[...]
        h = jnp.dot(a, w_ref[...], preferred_element_type=jnp.float32)
        h = jnp.maximum(h + b_ref[...], 0.0)
        if scale_out:
            h = h * ns_ref[...]
        o_ref[...] = h

    return pl.pallas_call(
        body,
        grid=(N // _R,),
        in_specs=[
            pl.BlockSpec((NC, _R, D), lambda i: (0, i, 0)),
            pl.BlockSpec((_R, 1), lambda i: (i, 0)),
            pl.BlockSpec((_R, 1), lambda i: (i, 0)),
            pl.BlockSpec((D, D), lambda i: (0, 0)),
            pl.BlockSpec((1, D), lambda i: (0, 0)),
        ],
        out_specs=pl.BlockSpec((_R, D), lambda i: (i, 0)),
        out_shape=jax.ShapeDtypeStruct((N, D), jnp.float32),
    )(agg, nd, ns, w, b)


def kernel(x, edge_index, W1, b1, W2, b2):
    src = edge_index[0].astype(jnp.int32)
    dst = edge_index[1].astype(jnp.int32)

    deg_out, deg_in = _sc_degrees(src, dst)
    xs, nsrc, ndst = _tc_norms(deg_out, deg_in, x)

    agg1 = _sc_aggregate(xs, src, dst)
    h1 = _tc_layer(agg1, ndst, nsrc, W1, b1.reshape(1, D), scale_out=True)

    agg2 = _sc_aggregate(h1, src, dst)
    out = _tc_layer(agg2, ndst, nsrc, W2, b2.reshape(1, D), scale_out=False)
    return out
